# baseline (speedup 1.0000x reference)
import functools

import jax
import jax.numpy as jnp
from jax import lax
from jax.experimental import pallas as pl
from jax.experimental.pallas import tpu as pltpu
from jax.experimental.pallas import tpu_sc as plsc

N = 2226
E = 6525
R = 1800
OUT = R + E + N + N

BASE_C = R
BASE_Q = R + E
BASE_S = R + E + N

NC = 2
NS = 16
NW = NC * NS

PER_R = 64
PER_E = 208
PER_N = 80

_F32 = jnp.float32
_I32 = jnp.int32


def _sqrt16(ss):
    i = lax.bitcast_convert_type(ss, _I32)
    y = lax.bitcast_convert_type(
        jnp.int32(0x5F3759DF) - lax.shift_right_logical(i, 1), _F32)
    for _ in range(3):
        y = y * (1.5 - 0.5 * ss * y * y)
    return jnp.where(ss > 0.0, ss * y, 0.0)


def _body(pos_h, str_h, rotm_h, foc_h, bias_h, dir_h, lene_h, act_h,
          rope_h, refl_h, edge_h,
          out_o,
          pos_v, act_v, dir_v, str_v, rope_v, refl_v, edge_v, lene_v,
          rotm_v, foc_v, bias_v,
          loss_s, c_s, ceq_s, stre_s, ixr_v, ixe_v, ixn_v, ixs_v, sem):
    wid = lax.axis_index("s") * NC + lax.axis_index("c")

    pairs = [(pos_h, pos_v), (act_h, act_v), (dir_h, dir_v),
             (str_h, str_v), (rope_h, rope_v), (refl_h, refl_v),
             (edge_h, edge_v), (lene_h, lene_v),
             (rotm_h, rotm_v.at[pl.ds(0, 9)]),
             (foc_h, foc_v.at[pl.ds(0, 3)]),
             (bias_h, bias_v.at[pl.ds(0, 1)])]
    handles = [pltpu.async_copy(src, dst, sem) for src, dst in pairs]
    for h in handles:
        h.wait()

    iota = lax.iota(_I32, 16)

    rv = rotm_v[...]
    fv = foc_v[...]
    bv = bias_v[...]
    r00, r01, r02 = rv[0], rv[1], rv[2]
    r10, r11, r12 = rv[3], rv[4], rv[5]
    r20, r21, r22 = rv[6], rv[7], rv[8]
    fx, fy, fz = fv[0], fv[1], fv[2]
    bias2 = bv[0] * 2.0 + 440.0

    base_r = wid * PER_R
    for j in range(PER_R // 16):
        ii = jnp.minimum(base_r + j * 16 + iota, R - 1)
        ridx = plsc.load_gather(refl_v, [ii]) * 3
        px = plsc.load_gather(pos_v, [ridx])
        py = plsc.load_gather(pos_v, [ridx + 1])
        pz = plsc.load_gather(pos_v, [ridx + 2])
        rx = px * r00 + py * r10 + pz * r20
        ry = px * r01 + py * r11 + pz * r21
        rz = px * r02 + py * r12 + pz * r22
        ex = rx - fx
        ey = ry - fy
        ez = rz - fz
        dis = _sqrt16(ex * ex + ey * ey + ez * ez)
        t = jnp.abs(dis - (rz + bias2)) - 1.0
        loss_s[pl.ds(j * 16, 16)] = jnp.maximum(t, 0.0)
        ixr_v[pl.ds(j * 16, 16)] = ii

    base_e = wid * PER_E
    for j in range(PER_E // 16):
        ii = jnp.minimum(base_e + j * 16 + iota, E - 1)
        ia = plsc.load_gather(edge_v, [ii * 2]) * 3
        ib = plsc.load_gather(edge_v, [ii * 2 + 1]) * 3
        dx = plsc.load_gather(pos_v, [ia]) - plsc.load_gather(pos_v, [ib])
        dy = (plsc.load_gather(pos_v, [ia + 1])
              - plsc.load_gather(pos_v, [ib + 1]))
        dz = (plsc.load_gather(pos_v, [ia + 2])
              - plsc.load_gather(pos_v, [ib + 2]))
        lens = _sqrt16(dx * dx + dy * dy + dz * dz)
        le = plsc.load_gather(lene_v, [ii])
        c = jnp.maximum(jnp.abs(lens - le) - 0.007 * le, 0.0) * 100.0
        c_s[pl.ds(j * 16, 16)] = c
        ixe_v[pl.ds(j * 16, 16)] = ii + BASE_C

    base_n = wid * PER_N
    for j in range(PER_N // 16):
        ii = jnp.minimum(base_n + j * 16 + iota, N - 1)
        i3 = ii * 3
        s = plsc.load_gather(str_v, [ii])
        rx = (plsc.load_gather(act_v, [i3])
              + plsc.load_gather(dir_v, [i3]) * s
              - plsc.load_gather(pos_v, [i3]))
        ry = (plsc.load_gather(act_v, [i3 + 1])
              + plsc.load_gather(dir_v, [i3 + 1]) * s
              - plsc.load_gather(pos_v, [i3 + 1]))
        rz = (plsc.load_gather(act_v, [i3 + 2])
              + plsc.load_gather(dir_v, [i3 + 2]) * s
              - plsc.load_gather(pos_v, [i3 + 2]))
        nn = _sqrt16(rx * rx + ry * ry + rz * rz)
        lr = plsc.load_gather(rope_v, [ii])
        ceq_s[pl.ds(j * 16, 16)] = jnp.abs(lr - nn) * 100.0
        stre_s[pl.ds(j * 16, 16)] = jnp.maximum(jnp.abs(s) - 0.6, 0.0)
        ixn_v[pl.ds(j * 16, 16)] = ii + BASE_Q
        ixs_v[pl.ds(j * 16, 16)] = ii + BASE_S

    hs = [pltpu.async_copy(loss_s, out_o.at[ixr_v], sem),
          pltpu.async_copy(c_s, out_o.at[ixe_v], sem),
          pltpu.async_copy(ceq_s, out_o.at[ixn_v], sem),
          pltpu.async_copy(stre_s, out_o.at[ixs_v], sem)]
    for h in hs:
        h.wait()


_sc_call = functools.partial(
    pl.kernel,
    out_type=[
        jax.ShapeDtypeStruct((OUT,), _F32),
    ],
    mesh=plsc.VectorSubcoreMesh(core_axis_name="c", subcore_axis_name="s",
                                num_cores=NC, num_subcores=NS),
    compiler_params=pltpu.CompilerParams(needs_layout_passes=False),
    scratch_types=[
        pltpu.VMEM((N * 3,), _F32),
        pltpu.VMEM((N * 3,), _F32),
        pltpu.VMEM((N * 3,), _F32),
        pltpu.VMEM((N,), _F32),
        pltpu.VMEM((N,), _F32),
        pltpu.VMEM((R,), _I32),
        pltpu.VMEM((E * 2,), _I32),
        pltpu.VMEM((E,), _F32),
        pltpu.VMEM((16,), _F32),
        pltpu.VMEM((16,), _F32),
        pltpu.VMEM((16,), _F32),
        pltpu.VMEM((PER_R,), _F32),
        pltpu.VMEM((PER_E,), _F32),
        pltpu.VMEM((PER_N,), _F32),
        pltpu.VMEM((PER_N,), _F32),
        pltpu.VMEM((PER_R,), _I32),
        pltpu.VMEM((PER_E,), _I32),
        pltpu.VMEM((PER_N,), _I32),
        pltpu.VMEM((PER_N,), _I32),
        pltpu.SemaphoreType.DMA,
    ],
)(_body)


def kernel(pos, stretch, bias, rotm, direction, focus, len_edges, act_up,
           len_rope, refl_idx, all_edges):
    (out,) = _sc_call(
        pos.reshape(-1), stretch.reshape(-1), rotm.reshape(-1), focus,
        bias, direction.reshape(-1), len_edges, act_up.reshape(-1),
        len_rope, refl_idx.astype(_I32),
        all_edges.astype(_I32).reshape(-1))
    return out


# speedup vs baseline: 3.7098x; 3.7098x over previous
import functools

import jax
import jax.numpy as jnp
from jax import lax
from jax.experimental import pallas as pl
from jax.experimental.pallas import tpu as pltpu
from jax.experimental.pallas import tpu_sc as plsc

N = 2226
E = 6525
R = 1800

NC = 2
NS = 16
NW = NC * NS

PER_R = 64
PER_E = 208
PER_N = 80

S_REFL = PER_R
S_LENE = PER_E + 5
S_EDGE = 2 * PER_E + 2
S_NODE3 = 3 * PER_N + 6
S_NODE = PER_N + 2

_F32 = jnp.float32
_I32 = jnp.int32


def _sqrt16(ss):
    i = lax.bitcast_convert_type(ss, _I32)
    y = lax.bitcast_convert_type(
        jnp.int32(0x5F3759DF) - lax.shift_right_logical(i, 1), _F32)
    for _ in range(3):
        y = y * (1.5 - 0.5 * ss * y * y)
    return jnp.where(ss > 0.0, ss * y, 0.0)


def _body(pos_h, str_h, rotm_h, foc_h, bias_h, dir_h, lene_h, act_h,
          rope_h, refl_h, edge_h,
          loss_o, c_o, ceq_o, stre_o,
          pos_v, act_v, dir_v, str_v, rope_v, refl_v, edge_v, lene_v,
          rotm_v, foc_v, bias_v,
          loss_s, c_s, ceq_s, stre_s, sem):
    wid = lax.axis_index("s") * NC + lax.axis_index("c")

    base_r = wid * PER_R
    base_e = wid * PER_E
    base_n = wid * PER_N

    b_refl = jnp.minimum(base_r, R - S_REFL)
    b_lene = jnp.minimum(base_e, E - S_LENE)
    b_edge = jnp.minimum(2 * base_e, 2 * E - S_EDGE)
    b_nod3 = jnp.minimum(3 * base_n, 3 * N - S_NODE3)
    b_node = jnp.minimum(base_n, N - S_NODE)

    pairs = [
        (pos_h, pos_v),
        (act_h.at[pl.ds(b_nod3, S_NODE3)], act_v),
        (dir_h.at[pl.ds(b_nod3, S_NODE3)], dir_v),
        (str_h.at[pl.ds(b_node, S_NODE)], str_v),
        (rope_h.at[pl.ds(b_node, S_NODE)], rope_v),
        (refl_h.at[pl.ds(b_refl, S_REFL)], refl_v),
        (edge_h.at[pl.ds(b_edge, S_EDGE)], edge_v),
        (lene_h.at[pl.ds(b_lene, S_LENE)], lene_v),
        (rotm_h, rotm_v.at[pl.ds(0, 9)]),
        (foc_h, foc_v.at[pl.ds(0, 3)]),
        (bias_h, bias_v.at[pl.ds(0, 1)]),
    ]
    handles = [pltpu.async_copy(src, dst, sem) for src, dst in pairs]
    for h in handles:
        h.wait()

    iota = lax.iota(_I32, 16)

    rv = rotm_v[...]
    fv = foc_v[...]
    bv = bias_v[...]
    r00, r01, r02 = rv[0], rv[1], rv[2]
    r10, r11, r12 = rv[3], rv[4], rv[5]
    r20, r21, r22 = rv[6], rv[7], rv[8]
    fx, fy, fz = fv[0], fv[1], fv[2]
    bias2 = bv[0] * 2.0 + 440.0

    for j in range(PER_R // 16):
        ii = jnp.minimum(base_r + j * 16 + iota, R - 1)
        ridx = plsc.load_gather(refl_v, [ii - b_refl]) * 3
        px = plsc.load_gather(pos_v, [ridx])
        py = plsc.load_gather(pos_v, [ridx + 1])
        pz = plsc.load_gather(pos_v, [ridx + 2])
        rx = px * r00 + py * r10 + pz * r20
        ry = px * r01 + py * r11 + pz * r21
        rz = px * r02 + py * r12 + pz * r22
        ex = rx - fx
        ey = ry - fy
        ez = rz - fz
        dis = _sqrt16(ex * ex + ey * ey + ez * ez)
        t = jnp.abs(dis - (rz + bias2)) - 1.0
        loss_s[pl.ds(j * 16, 16)] = jnp.maximum(t, 0.0)

    for j in range(PER_E // 16):
        ii = jnp.minimum(base_e + j * 16 + iota, E - 1)
        ia = plsc.load_gather(edge_v, [ii * 2 - b_edge]) * 3
        ib = plsc.load_gather(edge_v, [ii * 2 + 1 - b_edge]) * 3
        dx = plsc.load_gather(pos_v, [ia]) - plsc.load_gather(pos_v, [ib])
        dy = (plsc.load_gather(pos_v, [ia + 1])
              - plsc.load_gather(pos_v, [ib + 1]))
        dz = (plsc.load_gather(pos_v, [ia + 2])
              - plsc.load_gather(pos_v, [ib + 2]))
        lens = _sqrt16(dx * dx + dy * dy + dz * dz)
        le = plsc.load_gather(lene_v, [ii - b_lene])
        c = jnp.maximum(jnp.abs(lens - le) - 0.007 * le, 0.0) * 100.0
        c_s[pl.ds(j * 16, 16)] = c

    for j in range(PER_N // 16):
        ii = jnp.minimum(base_n + j * 16 + iota, N - 1)
        i3 = ii * 3 - b_nod3
        s = plsc.load_gather(str_v, [ii - b_node])
        rx = (plsc.load_gather(act_v, [i3])
              + plsc.load_gather(dir_v, [i3]) * s
              - plsc.load_gather(pos_v, [ii * 3]))
        ry = (plsc.load_gather(act_v, [i3 + 1])
              + plsc.load_gather(dir_v, [i3 + 1]) * s
              - plsc.load_gather(pos_v, [ii * 3 + 1]))
        rz = (plsc.load_gather(act_v, [i3 + 2])
              + plsc.load_gather(dir_v, [i3 + 2]) * s
              - plsc.load_gather(pos_v, [ii * 3 + 2]))
        nn = _sqrt16(rx * rx + ry * ry + rz * rz)
        lr = plsc.load_gather(rope_v, [ii - b_node])
        ceq_s[pl.ds(j * 16, 16)] = jnp.abs(lr - nn) * 100.0
        stre_s[pl.ds(j * 16, 16)] = jnp.maximum(jnp.abs(s) - 0.6, 0.0)

    pltpu.sync_copy(loss_s, loss_o.at[pl.ds(base_r, PER_R)])
    pltpu.sync_copy(c_s, c_o.at[pl.ds(base_e, PER_E)])
    pltpu.sync_copy(ceq_s, ceq_o.at[pl.ds(base_n, PER_N)])
    pltpu.sync_copy(stre_s, stre_o.at[pl.ds(base_n, PER_N)])


_sc_call = functools.partial(
    pl.kernel,
    out_type=[
        jax.ShapeDtypeStruct((NW * PER_R,), _F32),
        jax.ShapeDtypeStruct((NW * PER_E,), _F32),
        jax.ShapeDtypeStruct((NW * PER_N,), _F32),
        jax.ShapeDtypeStruct((NW * PER_N,), _F32),
    ],
    mesh=plsc.VectorSubcoreMesh(core_axis_name="c", subcore_axis_name="s",
                                num_cores=NC, num_subcores=NS),
    compiler_params=pltpu.CompilerParams(needs_layout_passes=False),
    scratch_types=[
        pltpu.VMEM((N * 3,), _F32),
        pltpu.VMEM((S_NODE3,), _F32),
        pltpu.VMEM((S_NODE3,), _F32),
        pltpu.VMEM((S_NODE,), _F32),
        pltpu.VMEM((S_NODE,), _F32),
        pltpu.VMEM((S_REFL,), _I32),
        pltpu.VMEM((S_EDGE,), _I32),
        pltpu.VMEM((S_LENE,), _F32),
        pltpu.VMEM((16,), _F32),
        pltpu.VMEM((16,), _F32),
        pltpu.VMEM((16,), _F32),
        pltpu.VMEM((PER_R,), _F32),
        pltpu.VMEM((PER_E,), _F32),
        pltpu.VMEM((PER_N,), _F32),
        pltpu.VMEM((PER_N,), _F32),
        pltpu.SemaphoreType.DMA,
    ],
)(_body)


def kernel(pos, stretch, bias, rotm, direction, focus, len_edges, act_up,
           len_rope, refl_idx, all_edges):
    loss_p, c_p, ceq_p, stre_p = _sc_call(
        pos.reshape(-1), stretch.reshape(-1), rotm.reshape(-1), focus,
        bias, direction.reshape(-1), len_edges, act_up.reshape(-1),
        len_rope, refl_idx.astype(_I32),
        all_edges.astype(_I32).reshape(-1))
    return jnp.concatenate([loss_p[:R], c_p[:E], ceq_p[:N], stre_p[:N]])


# speedup vs baseline: 3.8165x; 1.0287x over previous
import functools

import jax
import jax.numpy as jnp
from jax import lax
from jax.experimental import pallas as pl
from jax.experimental.pallas import tpu as pltpu
from jax.experimental.pallas import tpu_sc as plsc

N = 2226
E = 6525
R = 1800

NC = 2
NS = 16
NW = NC * NS

PER_R = 64
PER_E = 208
PER_N = 80

S_REFL = PER_R
S_LENE = PER_E + 5
S_EDGE = 2 * PER_E + 2
S_NODE3 = 3 * PER_N + 6
S_NODE = PER_N + 2


def _align8(x):
    return (x + 7) // 8 * 8


O_POS = 0
O_ACT = _align8(O_POS + 3 * N)
O_DIR = _align8(O_ACT + 3 * N)
O_STR = _align8(O_DIR + 3 * N)
O_ROPE = _align8(O_STR + N)
O_LENE = _align8(O_ROPE + N)
O_CONST = _align8(O_LENE + E)
F_TOTAL = O_CONST + 16

O_REFL = 0
O_EDGE = _align8(O_REFL + R)
I_TOTAL = O_EDGE + 2 * E

_F32 = jnp.float32
_I32 = jnp.int32


def _sqrt16(ss):
    i = lax.bitcast_convert_type(ss, _I32)
    y = lax.bitcast_convert_type(
        jnp.int32(0x5F3759DF) - lax.shift_right_logical(i, 1), _F32)
    for _ in range(3):
        y = y * (1.5 - 0.5 * ss * y * y)
    return jnp.where(ss > 0.0, ss * y, 0.0)


def _body(fbuf_h, ibuf_h,
          loss_o, c_o, ceq_o, stre_o,
          pos_v, act_v, dir_v, str_v, rope_v, refl_v, edge_v, lene_v,
          consts_v,
          loss_s, c_s, ceq_s, stre_s, sem):
    wid = lax.axis_index("s") * NC + lax.axis_index("c")

    base_r = wid * PER_R
    base_e = wid * PER_E
    base_n = wid * PER_N

    b_refl = jnp.minimum(base_r, R - S_REFL)
    b_lene = jnp.minimum(base_e, E - S_LENE)
    b_edge = jnp.minimum(2 * base_e, 2 * E - S_EDGE)
    b_nod3 = jnp.minimum(3 * base_n, 3 * N - S_NODE3)
    b_node = jnp.minimum(base_n, N - S_NODE)

    pairs = [
        (fbuf_h.at[pl.ds(O_POS, 3 * N)], pos_v),
        (fbuf_h.at[pl.ds(O_ACT + b_nod3, S_NODE3)], act_v),
        (fbuf_h.at[pl.ds(O_DIR + b_nod3, S_NODE3)], dir_v),
        (fbuf_h.at[pl.ds(O_STR + b_node, S_NODE)], str_v),
        (fbuf_h.at[pl.ds(O_ROPE + b_node, S_NODE)], rope_v),
        (fbuf_h.at[pl.ds(O_LENE + b_lene, S_LENE)], lene_v),
        (fbuf_h.at[pl.ds(O_CONST, 16)], consts_v),
        (ibuf_h.at[pl.ds(O_REFL + b_refl, S_REFL)], refl_v),
        (ibuf_h.at[pl.ds(O_EDGE + b_edge, S_EDGE)], edge_v),
    ]
    handles = [pltpu.async_copy(src, dst, sem) for src, dst in pairs]
    for h in handles:
        h.wait()

    iota = lax.iota(_I32, 16)

    cv = consts_v[...]
    r00, r01, r02 = cv[0], cv[1], cv[2]
    r10, r11, r12 = cv[3], cv[4], cv[5]
    r20, r21, r22 = cv[6], cv[7], cv[8]
    fx, fy, fz = cv[9], cv[10], cv[11]
    bias2 = cv[12] * 2.0 + 440.0

    for j in range(PER_R // 16):
        ii = jnp.minimum(base_r + j * 16 + iota, R - 1)
        ridx = plsc.load_gather(refl_v, [ii - b_refl]) * 3
        px = plsc.load_gather(pos_v, [ridx])
        py = plsc.load_gather(pos_v, [ridx + 1])
        pz = plsc.load_gather(pos_v, [ridx + 2])
        rx = px * r00 + py * r10 + pz * r20
        ry = px * r01 + py * r11 + pz * r21
        rz = px * r02 + py * r12 + pz * r22
        ex = rx - fx
        ey = ry - fy
        ez = rz - fz
        dis = _sqrt16(ex * ex + ey * ey + ez * ez)
        t = jnp.abs(dis - (rz + bias2)) - 1.0
        loss_s[pl.ds(j * 16, 16)] = jnp.maximum(t, 0.0)

    for j in range(PER_E // 16):
        ii = jnp.minimum(base_e + j * 16 + iota, E - 1)
        ia = plsc.load_gather(edge_v, [ii * 2 - b_edge]) * 3
        ib = plsc.load_gather(edge_v, [ii * 2 + 1 - b_edge]) * 3
        dx = plsc.load_gather(pos_v, [ia]) - plsc.load_gather(pos_v, [ib])
        dy = (plsc.load_gather(pos_v, [ia + 1])
              - plsc.load_gather(pos_v, [ib + 1]))
        dz = (plsc.load_gather(pos_v, [ia + 2])
              - plsc.load_gather(pos_v, [ib + 2]))
        lens = _sqrt16(dx * dx + dy * dy + dz * dz)
        le = plsc.load_gather(lene_v, [ii - b_lene])
        c = jnp.maximum(jnp.abs(lens - le) - 0.007 * le, 0.0) * 100.0
        c_s[pl.ds(j * 16, 16)] = c

    for j in range(PER_N // 16):
        ii = jnp.minimum(base_n + j * 16 + iota, N - 1)
        i3 = ii * 3 - b_nod3
        s = plsc.load_gather(str_v, [ii - b_node])
        rx = (plsc.load_gather(act_v, [i3])
              + plsc.load_gather(dir_v, [i3]) * s
              - plsc.load_gather(pos_v, [ii * 3]))
        ry = (plsc.load_gather(act_v, [i3 + 1])
              + plsc.load_gather(dir_v, [i3 + 1]) * s
              - plsc.load_gather(pos_v, [ii * 3 + 1]))
        rz = (plsc.load_gather(act_v, [i3 + 2])
              + plsc.load_gather(dir_v, [i3 + 2]) * s
              - plsc.load_gather(pos_v, [ii * 3 + 2]))
        nn = _sqrt16(rx * rx + ry * ry + rz * rz)
        lr = plsc.load_gather(rope_v, [ii - b_node])
        ceq_s[pl.ds(j * 16, 16)] = jnp.abs(lr - nn) * 100.0
        stre_s[pl.ds(j * 16, 16)] = jnp.maximum(jnp.abs(s) - 0.6, 0.0)

    pltpu.sync_copy(loss_s, loss_o.at[pl.ds(base_r, PER_R)])
    pltpu.sync_copy(c_s, c_o.at[pl.ds(base_e, PER_E)])
    pltpu.sync_copy(ceq_s, ceq_o.at[pl.ds(base_n, PER_N)])
    pltpu.sync_copy(stre_s, stre_o.at[pl.ds(base_n, PER_N)])


_sc_call = functools.partial(
    pl.kernel,
    out_type=[
        jax.ShapeDtypeStruct((NW * PER_R,), _F32),
        jax.ShapeDtypeStruct((NW * PER_E,), _F32),
        jax.ShapeDtypeStruct((NW * PER_N,), _F32),
        jax.ShapeDtypeStruct((NW * PER_N,), _F32),
    ],
    mesh=plsc.VectorSubcoreMesh(core_axis_name="c", subcore_axis_name="s",
                                num_cores=NC, num_subcores=NS),
    compiler_params=pltpu.CompilerParams(needs_layout_passes=False),
    scratch_types=[
        pltpu.VMEM((N * 3,), _F32),
        pltpu.VMEM((S_NODE3,), _F32),
        pltpu.VMEM((S_NODE3,), _F32),
        pltpu.VMEM((S_NODE,), _F32),
        pltpu.VMEM((S_NODE,), _F32),
        pltpu.VMEM((S_REFL,), _I32),
        pltpu.VMEM((S_EDGE,), _I32),
        pltpu.VMEM((S_LENE,), _F32),
        pltpu.VMEM((16,), _F32),
        pltpu.VMEM((PER_R,), _F32),
        pltpu.VMEM((PER_E,), _F32),
        pltpu.VMEM((PER_N,), _F32),
        pltpu.VMEM((PER_N,), _F32),
        pltpu.SemaphoreType.DMA,
    ],
)(_body)


def _zpad(k):
    return jnp.zeros((k,), _F32)


def kernel(pos, stretch, bias, rotm, direction, focus, len_edges, act_up,
           len_rope, refl_idx, all_edges):
    fbuf = jnp.concatenate([
        pos.reshape(-1), _zpad(O_ACT - (O_POS + 3 * N)),
        act_up.reshape(-1), _zpad(O_DIR - (O_ACT + 3 * N)),
        direction.reshape(-1), _zpad(O_STR - (O_DIR + 3 * N)),
        stretch.reshape(-1), _zpad(O_ROPE - (O_STR + N)),
        len_rope, _zpad(O_LENE - (O_ROPE + N)),
        len_edges, _zpad(O_CONST - (O_LENE + E)),
        rotm.reshape(-1), focus, bias.reshape(1), _zpad(3),
    ])
    ibuf = jnp.concatenate([
        refl_idx.astype(_I32),
        jnp.zeros((O_EDGE - R,), _I32),
        all_edges.astype(_I32).reshape(-1),
    ])
    loss_p, c_p, ceq_p, stre_p = _sc_call(fbuf, ibuf)
    return jnp.concatenate([loss_p[:R], c_p[:E], ceq_p[:N], stre_p[:N]])


# speedup vs baseline: 3.8894x; 1.0191x over previous
import functools

import jax
import jax.numpy as jnp
from jax import lax
from jax.experimental import pallas as pl
from jax.experimental.pallas import tpu as pltpu
from jax.experimental.pallas import tpu_sc as plsc

N = 2226
E = 6525
R = 1800

NC = 2
NS = 16
NW = NC * NS

PER_R = 64
PER_E = 208
PER_N = 80

S_REFL = PER_R
S_LENE = PER_E + 5
S_EDGE = 2 * PER_E + 2
S_NODE3 = 3 * PER_N + 6
S_NODE = PER_N + 2


def _align8(x):
    return (x + 7) // 8 * 8


O_POS = 0
O_ACT = _align8(O_POS + 3 * N)
O_DIR = _align8(O_ACT + 3 * N)
O_STR = _align8(O_DIR + 3 * N)
O_ROPE = _align8(O_STR + N)
O_LENE = _align8(O_ROPE + N)
O_CONST = _align8(O_LENE + E)
F_TOTAL = O_CONST + 16

O_REFL = 0
O_EDGE = _align8(O_REFL + R)
I_TOTAL = O_EDGE + 2 * E

_F32 = jnp.float32
_I32 = jnp.int32


def _sqrt16(ss):
    i = lax.bitcast_convert_type(ss, _I32)
    y = lax.bitcast_convert_type(
        jnp.int32(0x5F3759DF) - lax.shift_right_logical(i, 1), _F32)
    for _ in range(3):
        y = y * (1.5 - 0.5 * ss * y * y)
    return jnp.where(ss > 0.0, ss * y, 0.0)


def _body(fbuf_h, ibuf_h,
          loss_o, c_o, ceq_o, stre_o,
          pos_v, act_v, dir_v, str_v, rope_v, refl_v, edge_v, lene_v,
          consts_v,
          loss_s, c_s, ceq_s, stre_s, sem):
    wid = lax.axis_index("s") * NC + lax.axis_index("c")

    base_r = wid * PER_R
    base_e = wid * PER_E
    base_n = wid * PER_N

    b_refl = jnp.minimum(base_r, R - S_REFL)
    b_lene = jnp.minimum(base_e, E - S_LENE)
    b_edge = jnp.minimum(2 * base_e, 2 * E - S_EDGE)
    b_nod3 = jnp.minimum(3 * base_n, 3 * N - S_NODE3)
    b_node = jnp.minimum(base_n, N - S_NODE)

    pairs = [
        (fbuf_h.at[pl.ds(O_POS, 3 * N)], pos_v),
        (fbuf_h.at[pl.ds(O_ACT + b_nod3, S_NODE3)], act_v),
        (fbuf_h.at[pl.ds(O_DIR + b_nod3, S_NODE3)], dir_v),
        (fbuf_h.at[pl.ds(O_STR + b_node, S_NODE)], str_v),
        (fbuf_h.at[pl.ds(O_ROPE + b_node, S_NODE)], rope_v),
        (fbuf_h.at[pl.ds(O_LENE + b_lene, S_LENE)], lene_v),
        (fbuf_h.at[pl.ds(O_CONST, 16)], consts_v),
        (ibuf_h.at[pl.ds(O_REFL + b_refl, S_REFL)], refl_v),
        (ibuf_h.at[pl.ds(O_EDGE + b_edge, S_EDGE)], edge_v),
    ]
    handles = [pltpu.async_copy(src, dst, sem) for src, dst in pairs]
    for h in handles:
        h.wait()

    iota = lax.iota(_I32, 16)

    cv = consts_v[...]
    r00, r01, r02 = cv[0], cv[1], cv[2]
    r10, r11, r12 = cv[3], cv[4], cv[5]
    r20, r21, r22 = cv[6], cv[7], cv[8]
    fx, fy, fz = cv[9], cv[10], cv[11]
    bias2 = cv[12] * 2.0 + 440.0

    def _loss_chunk(j, _):
        ii = jnp.minimum(base_r + j * 16 + iota, R - 1)
        ridx = plsc.load_gather(refl_v, [ii - b_refl]) * 3
        px = plsc.load_gather(pos_v, [ridx])
        py = plsc.load_gather(pos_v, [ridx + 1])
        pz = plsc.load_gather(pos_v, [ridx + 2])
        rx = px * r00 + py * r10 + pz * r20
        ry = px * r01 + py * r11 + pz * r21
        rz = px * r02 + py * r12 + pz * r22
        ex = rx - fx
        ey = ry - fy
        ez = rz - fz
        dis = _sqrt16(ex * ex + ey * ey + ez * ez)
        t = jnp.abs(dis - (rz + bias2)) - 1.0
        loss_s[pl.ds(j * 16, 16)] = jnp.maximum(t, 0.0)
        return 0

    lax.fori_loop(0, PER_R // 16, _loss_chunk, 0, unroll=False)

    def _edge_chunk(j, _):
        ii = jnp.minimum(base_e + j * 16 + iota, E - 1)
        ia = plsc.load_gather(edge_v, [ii * 2 - b_edge]) * 3
        ib = plsc.load_gather(edge_v, [ii * 2 + 1 - b_edge]) * 3
        dx = plsc.load_gather(pos_v, [ia]) - plsc.load_gather(pos_v, [ib])
        dy = (plsc.load_gather(pos_v, [ia + 1])
              - plsc.load_gather(pos_v, [ib + 1]))
        dz = (plsc.load_gather(pos_v, [ia + 2])
              - plsc.load_gather(pos_v, [ib + 2]))
        lens = _sqrt16(dx * dx + dy * dy + dz * dz)
        le = plsc.load_gather(lene_v, [ii - b_lene])
        c = jnp.maximum(jnp.abs(lens - le) - 0.007 * le, 0.0) * 100.0
        c_s[pl.ds(j * 16, 16)] = c
        return 0

    lax.fori_loop(0, PER_E // 16, _edge_chunk, 0, unroll=False)

    def _node_chunk(j, _):
        ii = jnp.minimum(base_n + j * 16 + iota, N - 1)
        i3 = ii * 3 - b_nod3
        s = plsc.load_gather(str_v, [ii - b_node])
        rx = (plsc.load_gather(act_v, [i3])
              + plsc.load_gather(dir_v, [i3]) * s
              - plsc.load_gather(pos_v, [ii * 3]))
        ry = (plsc.load_gather(act_v, [i3 + 1])
              + plsc.load_gather(dir_v, [i3 + 1]) * s
              - plsc.load_gather(pos_v, [ii * 3 + 1]))
        rz = (plsc.load_gather(act_v, [i3 + 2])
              + plsc.load_gather(dir_v, [i3 + 2]) * s
              - plsc.load_gather(pos_v, [ii * 3 + 2]))
        nn = _sqrt16(rx * rx + ry * ry + rz * rz)
        lr = plsc.load_gather(rope_v, [ii - b_node])
        ceq_s[pl.ds(j * 16, 16)] = jnp.abs(lr - nn) * 100.0
        stre_s[pl.ds(j * 16, 16)] = jnp.maximum(jnp.abs(s) - 0.6, 0.0)
        return 0

    lax.fori_loop(0, PER_N // 16, _node_chunk, 0, unroll=False)

    pltpu.sync_copy(loss_s, loss_o.at[pl.ds(base_r, PER_R)])
    pltpu.sync_copy(c_s, c_o.at[pl.ds(base_e, PER_E)])
    pltpu.sync_copy(ceq_s, ceq_o.at[pl.ds(base_n, PER_N)])
    pltpu.sync_copy(stre_s, stre_o.at[pl.ds(base_n, PER_N)])


_sc_call = functools.partial(
    pl.kernel,
    out_type=[
        jax.ShapeDtypeStruct((NW * PER_R,), _F32),
        jax.ShapeDtypeStruct((NW * PER_E,), _F32),
        jax.ShapeDtypeStruct((NW * PER_N,), _F32),
        jax.ShapeDtypeStruct((NW * PER_N,), _F32),
    ],
    mesh=plsc.VectorSubcoreMesh(core_axis_name="c", subcore_axis_name="s",
                                num_cores=NC, num_subcores=NS),
    compiler_params=pltpu.CompilerParams(needs_layout_passes=False),
    scratch_types=[
        pltpu.VMEM((N * 3,), _F32),
        pltpu.VMEM((S_NODE3,), _F32),
        pltpu.VMEM((S_NODE3,), _F32),
        pltpu.VMEM((S_NODE,), _F32),
        pltpu.VMEM((S_NODE,), _F32),
        pltpu.VMEM((S_REFL,), _I32),
        pltpu.VMEM((S_EDGE,), _I32),
        pltpu.VMEM((S_LENE,), _F32),
        pltpu.VMEM((16,), _F32),
        pltpu.VMEM((PER_R,), _F32),
        pltpu.VMEM((PER_E,), _F32),
        pltpu.VMEM((PER_N,), _F32),
        pltpu.VMEM((PER_N,), _F32),
        pltpu.SemaphoreType.DMA,
    ],
)(_body)


def _zpad(k):
    return jnp.zeros((k,), _F32)


def kernel(pos, stretch, bias, rotm, direction, focus, len_edges, act_up,
           len_rope, refl_idx, all_edges):
    fbuf = jnp.concatenate([
        pos.reshape(-1), _zpad(O_ACT - (O_POS + 3 * N)),
        act_up.reshape(-1), _zpad(O_DIR - (O_ACT + 3 * N)),
        direction.reshape(-1), _zpad(O_STR - (O_DIR + 3 * N)),
        stretch.reshape(-1), _zpad(O_ROPE - (O_STR + N)),
        len_rope, _zpad(O_LENE - (O_ROPE + N)),
        len_edges, _zpad(O_CONST - (O_LENE + E)),
        rotm.reshape(-1), focus, bias.reshape(1), _zpad(3),
    ])
    ibuf = jnp.concatenate([
        refl_idx.astype(_I32),
        jnp.zeros((O_EDGE - R,), _I32),
        all_edges.astype(_I32).reshape(-1),
    ])
    loss_p, c_p, ceq_p, stre_p = _sc_call(fbuf, ibuf)
    return jnp.concatenate([loss_p[:R], c_p[:E], ceq_p[:N], stre_p[:N]])
